# baseline (speedup 1.0000x reference)
import functools

import jax
import jax.numpy as jnp
import numpy as np
from jax import lax
from jax.experimental import pallas as pl
from jax.experimental.pallas import tpu as pltpu
from jax.experimental.pallas import tpu_sc as plsc

N = 10000
NPAD = 10112
E = 320000
EPAD = 331776
NW = 32
ET = EPAD // NW
B = 96
NBLK = ET // B
RPT = NPAD // 16
RBLK = 1264

_NEG = -1e30


def _front_body(x_ref, w_ref, ha_ref, add_ref, *, dout):
    i = pl.program_id(0)
    hb = jnp.dot(x_ref[...], w_ref[...], preferred_element_type=jnp.float32)
    rows = i * RBLK + lax.broadcasted_iota(jnp.int32, (RBLK, 16), 0)
    mask = rows < N
    ha_ref[:, :dout] = hb[:, :dout]
    ha_ref[:, dout:dout + 16] = jnp.where(mask, hb[:, dout:dout + 16], _NEG)
    add_ref[...] = jnp.where(mask, hb[:, dout + 16:dout + 32], _NEG)


def _mid_body(p_ref, bias_ref, w_ref, rep_ref, ha_ref, add_ref,
              *, din, dout):
    i = pl.program_id(0)
    pa = p_ref[0] + p_ref[1]
    p = pa[:, :din]
    s16 = jnp.maximum(pa[:, din:din + 16], 1e-30)
    rex = jnp.dot(1.0 / s16, rep_ref[...], preferred_element_type=jnp.float32)
    xn = p * rex + bias_ref[...]
    xn = jnp.where(xn > 0, xn, jnp.exp(jnp.minimum(xn, 0.0)) - 1.0)
    hb = jnp.dot(xn, w_ref[...], preferred_element_type=jnp.float32)
    rows = i * RBLK + lax.broadcasted_iota(jnp.int32, (RBLK, 16), 0)
    mask = rows < N
    ha_ref[:, :dout] = hb[:, :dout]
    ha_ref[:, dout:dout + 16] = jnp.where(mask, hb[:, dout:dout + 16], _NEG)
    add_ref[...] = jnp.where(mask, hb[:, dout + 16:dout + 32], _NEG)


def _fin_body(p_ref, bias_ref, rep_ref, o_ref):
    pa = p_ref[0] + p_ref[1]
    s16 = jnp.maximum(pa[:, 64:80], 1e-30)
    rex = jnp.dot(1.0 / s16, rep_ref[...], preferred_element_type=jnp.float32)
    o_ref[...] = pa[:, :64] * rex + bias_ref[...]


def _tc_front(xpad, wbig, dout):
    return pl.pallas_call(
        functools.partial(_front_body, dout=dout),
        grid=(NPAD // RBLK,),
        in_specs=[
            pl.BlockSpec((RBLK, 128), lambda i: (i, 0)),
            pl.BlockSpec(wbig.shape, lambda i: (0, 0)),
        ],
        out_specs=[
            pl.BlockSpec((RBLK, dout + 16), lambda i: (i, 0)),
            pl.BlockSpec((RBLK, 16), lambda i: (i, 0)),
        ],
        out_shape=[
            jax.ShapeDtypeStruct((NPAD, dout + 16), jnp.float32),
            jax.ShapeDtypeStruct((NPAD, 16), jnp.float32),
        ],
    )(xpad, wbig)


def _tc_mid(P, bias, wbig, rep, din, dout):
    return pl.pallas_call(
        functools.partial(_mid_body, din=din, dout=dout),
        grid=(NPAD // RBLK,),
        in_specs=[
            pl.BlockSpec((2, RBLK, din + 16), lambda i: (0, i, 0)),
            pl.BlockSpec((1, din), lambda i: (0, 0)),
            pl.BlockSpec(wbig.shape, lambda i: (0, 0)),
            pl.BlockSpec((16, din), lambda i: (0, 0)),
        ],
        out_specs=[
            pl.BlockSpec((RBLK, dout + 16), lambda i: (i, 0)),
            pl.BlockSpec((RBLK, 16), lambda i: (i, 0)),
        ],
        out_shape=[
            jax.ShapeDtypeStruct((NPAD, dout + 16), jnp.float32),
            jax.ShapeDtypeStruct((NPAD, 16), jnp.float32),
        ],
    )(P, bias, wbig, rep)


def _tc_fin(P, bias, rep):
    return pl.pallas_call(
        _fin_body,
        grid=(NPAD // RBLK,),
        in_specs=[
            pl.BlockSpec((2, RBLK, 80), lambda i: (0, i, 0)),
            pl.BlockSpec((1, 64), lambda i: (0, 0)),
            pl.BlockSpec((16, 64), lambda i: (0, 0)),
        ],
        out_specs=pl.BlockSpec((RBLK, 64), lambda i: (i, 0)),
        out_shape=jax.ShapeDtypeStruct((N, 64), jnp.float32),
    )(P, bias, rep)


_GDN = lax.GatherDimensionNumbers(
    offset_dims=(), collapsed_slice_dims=(0,), start_index_map=(0,))


def _lane_splat(v, k):
    idx = jnp.full((16, 1), k, jnp.int32)
    return lax.gather(v, idx, _GDN, (1,),
                      mode=lax.GatherScatterMode.PROMISE_IN_BOUNDS)


def _make_sc_edge(D, heads):
    chunks = D // 16
    DA = D + 16
    mesh = plsc.VectorSubcoreMesh(core_axis_name="c", subcore_axis_name="s",
                                  num_cores=2, num_subcores=16)

    @functools.partial(
        pl.kernel,
        out_type=jax.ShapeDtypeStruct((2, NPAD, D + 16), jnp.float32),
        mesh=mesh,
        compiler_params=pltpu.CompilerParams(use_tc_tiling_on_sc=False),
        scratch_types=(
            [pltpu.VMEM((2, B), jnp.int32)] * 4
            + [pltpu.VMEM((B, 16), jnp.float32)] * 2
            + [pltpu.VMEM((B, D + 16), jnp.float32)] * 2
            + [
                pltpu.VMEM((8, D + 16), jnp.float32),
                pltpu.VMEM_SHARED((NPAD, D + 16), jnp.float32),
            ]
            + [pltpu.SemaphoreType.DMA] * 8
        ),
    )
    def k(ha_hbm, add_hbm, sd_hbm, p_out,
          sd0, sd1, sd2, sd3,
          add0, add1, ha0, ha1, zb, acc,
          gsem0, gsem1, ssem0, ssem1, isem0, isem1, isem2, isem3):
        sd_v = (sd0, sd1, sd2, sd3)
        src_v = tuple(r.at[0] for r in sd_v)
        dst_v = tuple(r.at[1] for r in sd_v)
        add_v = (add0, add1)
        ha_v = (ha0, ha1)
        gsem, ssem = (gsem0, gsem1), (ssem0, ssem1)
        isem = (isem0, isem1, isem2, isem3)

        c = lax.axis_index("c")
        s = lax.axis_index("s")
        w = c * 16 + s

        zero = jnp.zeros((16,), jnp.float32)
        for i in range(8):
            for j in range(chunks + 1):
                zb[i, pl.ds(j * 16, 16)] = zero
        r0 = s * RPT

        def zrow(i, carry):
            pltpu.sync_copy(zb, acc.at[pl.ds(r0 + i * 8, 8)])
            return carry

        lax.fori_loop(0, RPT // 8, zrow, 0)
        plsc.subcore_barrier()

        base_w = w * ET

        def issue_idx(r, blk_idx):
            base = base_w + blk_idx * B
            pltpu.async_copy(sd_hbm.at[:, pl.ds(base, B)], sd_v[r], isem[r])

        def wait_idx(r):
            pltpu.make_async_copy(sd_hbm.at[:, pl.ds(0, B)], sd_v[r],
                                  isem[r]).wait()

        def issue_gathers(g, r):
            pltpu.async_copy(add_hbm.at[dst_v[r]], add_v[g], gsem[g])
            pltpu.async_copy(ha_hbm.at[src_v[r]], ha_v[g], gsem[g])

        def wait_gathers(g):
            pltpu.make_async_copy(add_hbm.at[dst_v[0]], add_v[g],
                                  gsem[g]).wait()
            pltpu.make_async_copy(ha_hbm.at[src_v[0]], ha_v[g], gsem[g]).wait()

        def issue_scatter(g, r):
            pltpu.async_copy(ha_v[g], acc.at[dst_v[r]], ssem[g], add=True)

        def wait_scatter(g):
            pltpu.make_async_copy(ha_v[g], acc.at[dst_v[0]], ssem[g]).wait()

        issue_idx(0, jnp.int32(0))
        wait_idx(0)
        issue_idx(1, jnp.int32(1))
        issue_gathers(0, 0)
        NB4 = NBLK // 4

        def blk4(jj, carry):
            for p in (0, 1, 2, 3):
                j = 4 * jj + p
                g = p % 2
                o = 1 - g
                r_cur = p
                r_nxt = (p + 1) % 4
                r_pf = (p + 2) % 4

                if p == 0:
                    @pl.when(jj >= 1)
                    def _():
                        wait_scatter(o)
                else:
                    wait_scatter(o)
                if p >= 2:
                    @pl.when(jj < NB4 - 1)
                    def _():
                        issue_idx(r_pf, j + 2)
                else:
                    issue_idx(r_pf, j + 2)
                if p == 3:
                    @pl.when(jj < NB4 - 1)
                    def _():
                        wait_idx(r_nxt)
                        issue_gathers(o, r_nxt)
                else:
                    wait_idx(r_nxt)
                    issue_gathers(o, r_nxt)
                wait_gathers(g)

                @plsc.parallel_loop(0, B, unroll=2)
                def edge(b):
                    e = ha_v[g][b, pl.ds(D, 16)] + add_v[g][b, :]
                    ex = jnp.exp(jnp.maximum(e, 0.2 * e))
                    ha_v[g][b, pl.ds(D, 16)] = ex
                    for kk in range(chunks):
                        if heads == 8:
                            bc = _lane_splat(ex, kk)
                        else:
                            bc = ex
                        ha_v[g][b, pl.ds(kk * 16, 16)] = (
                            ha_v[g][b, pl.ds(kk * 16, 16)] * bc)

                issue_scatter(g, r_cur)
            return carry

        lax.fori_loop(0, NB4, blk4, 0)
        wait_scatter(1)
        plsc.subcore_barrier()
        pltpu.sync_copy(acc.at[pl.ds(r0, RPT)], p_out.at[c, pl.ds(r0, RPT)])

    return k


_sc_edge_128 = _make_sc_edge(128, 8)
_sc_edge_64 = _make_sc_edge(64, 1)


_Z8 = np.zeros((128, 16), np.float32)
for _k in range(8):
    for _c in range(16):
        _Z8[_k * 16 + _c, _k] = 1.0
        _Z8[_k * 16 + _c, _k + 8] = 1.0
_Z1 = np.ones((64, 16), np.float32)

_REP8 = np.zeros((16, 128), np.float32)
for _k in range(8):
    _REP8[_k, _k * 16:(_k + 1) * 16] = 0.5
    _REP8[_k + 8, _k * 16:(_k + 1) * 16] = 0.5
_REP1 = np.full((16, 64), 1.0 / 16.0, np.float32)


def _fold(W, a_src, a_dst, z):
    ms = z * a_src.reshape(-1)[:, None]
    md = z * a_dst.reshape(-1)[:, None]
    return jnp.concatenate([W, W @ ms, W @ md], axis=1)


def kernel(x, edge_index, W1, a_src1, a_dst1, b1, W2, a_src2, a_dst2, b2,
           W3, a_src3, a_dst3, b3):
    loop = jnp.arange(N, dtype=jnp.int32)
    padi = jnp.full((EPAD - E - N,), N, jnp.int32)
    srcp = jnp.concatenate([edge_index[0], loop, padi])
    dstp = jnp.concatenate([edge_index[1], loop, padi])
    sd = jnp.stack([srcp, dstp])
    xpad = jnp.pad(x, ((0, NPAD - N), (0, 0)))

    HA, ADD = _tc_front(xpad, _fold(W1, a_src1, a_dst1, _Z8), 128)
    P = _sc_edge_128(HA, ADD, sd)
    HA, ADD = _tc_mid(P, b1.reshape(1, -1),
                      _fold(W2, a_src2, a_dst2, _Z8), _REP8, 128, 128)
    P = _sc_edge_128(HA, ADD, sd)
    HA, ADD = _tc_mid(P, b2.reshape(1, -1),
                      _fold(W3, a_src3, a_dst3, _Z1), _REP8, 128, 64)
    P = _sc_edge_64(HA, ADD, sd)
    return _tc_fin(P, b3.reshape(1, -1), _REP1)


# speedup vs baseline: 1.0781x; 1.0781x over previous
import functools

import jax
import jax.numpy as jnp
import numpy as np
from jax import lax
from jax.experimental import pallas as pl
from jax.experimental.pallas import tpu as pltpu
from jax.experimental.pallas import tpu_sc as plsc

N = 10000
NPAD = 10112
E = 320000
EPAD = 331776
NW = 32
ET = EPAD // NW
B = 96
NBLK = ET // B
RPT = NPAD // 16
RBLK = 1264

_NEG = -1e30


def _front_body(x_ref, w_ref, h_ref, asd_ref, add_ref, *, dout):
    i = pl.program_id(0)
    hb = jnp.dot(x_ref[...], w_ref[...], preferred_element_type=jnp.float32)
    h_ref[...] = hb[:, :dout]
    rows = i * RBLK + lax.broadcasted_iota(jnp.int32, (RBLK, 16), 0)
    mask = rows < N
    asd_ref[...] = jnp.where(mask, hb[:, dout:dout + 16], _NEG)
    add_ref[...] = jnp.where(mask, hb[:, dout + 16:dout + 32], _NEG)


def _mid_body(p_ref, s_ref, bias_ref, w_ref, rep_ref, h_ref, asd_ref, add_ref,
              *, dout):
    i = pl.program_id(0)
    p = p_ref[0] + p_ref[1]
    s16 = jnp.maximum(s_ref[0] + s_ref[1], 1e-30)
    rex = jnp.dot(1.0 / s16, rep_ref[...], preferred_element_type=jnp.float32)
    xn = p * rex + bias_ref[...]
    xn = jnp.where(xn > 0, xn, jnp.exp(jnp.minimum(xn, 0.0)) - 1.0)
    hb = jnp.dot(xn, w_ref[...], preferred_element_type=jnp.float32)
    h_ref[...] = hb[:, :dout]
    rows = i * RBLK + lax.broadcasted_iota(jnp.int32, (RBLK, 16), 0)
    mask = rows < N
    asd_ref[...] = jnp.where(mask, hb[:, dout:dout + 16], _NEG)
    add_ref[...] = jnp.where(mask, hb[:, dout + 16:dout + 32], _NEG)


def _fin_body(p_ref, s_ref, bias_ref, rep_ref, o_ref):
    p = p_ref[0] + p_ref[1]
    s16 = jnp.maximum(s_ref[0] + s_ref[1], 1e-30)
    rex = jnp.dot(1.0 / s16, rep_ref[...], preferred_element_type=jnp.float32)
    o_ref[...] = p * rex + bias_ref[...]


def _tc_front(xpad, wbig, dout):
    return pl.pallas_call(
        functools.partial(_front_body, dout=dout),
        grid=(NPAD // RBLK,),
        in_specs=[
            pl.BlockSpec((RBLK, 128), lambda i: (i, 0)),
            pl.BlockSpec(wbig.shape, lambda i: (0, 0)),
        ],
        out_specs=[
            pl.BlockSpec((RBLK, dout), lambda i: (i, 0)),
            pl.BlockSpec((RBLK, 16), lambda i: (i, 0)),
            pl.BlockSpec((RBLK, 16), lambda i: (i, 0)),
        ],
        out_shape=[
            jax.ShapeDtypeStruct((NPAD, dout), jnp.float32),
            jax.ShapeDtypeStruct((NPAD, 16), jnp.float32),
            jax.ShapeDtypeStruct((NPAD, 16), jnp.float32),
        ],
    )(xpad, wbig)


def _tc_mid(P, S, bias, wbig, rep, din, dout):
    return pl.pallas_call(
        functools.partial(_mid_body, dout=dout),
        grid=(NPAD // RBLK,),
        in_specs=[
            pl.BlockSpec((2, RBLK, din), lambda i: (0, i, 0)),
            pl.BlockSpec((2, RBLK, 16), lambda i: (0, i, 0)),
            pl.BlockSpec((1, din), lambda i: (0, 0)),
            pl.BlockSpec(wbig.shape, lambda i: (0, 0)),
            pl.BlockSpec((16, din), lambda i: (0, 0)),
        ],
        out_specs=[
            pl.BlockSpec((RBLK, dout), lambda i: (i, 0)),
            pl.BlockSpec((RBLK, 16), lambda i: (i, 0)),
            pl.BlockSpec((RBLK, 16), lambda i: (i, 0)),
        ],
        out_shape=[
            jax.ShapeDtypeStruct((NPAD, dout), jnp.float32),
            jax.ShapeDtypeStruct((NPAD, 16), jnp.float32),
            jax.ShapeDtypeStruct((NPAD, 16), jnp.float32),
        ],
    )(P, S, bias, wbig, rep)


def _tc_fin(P, S, bias, rep, din):
    return pl.pallas_call(
        _fin_body,
        grid=(NPAD // RBLK,),
        in_specs=[
            pl.BlockSpec((2, RBLK, din), lambda i: (0, i, 0)),
            pl.BlockSpec((2, RBLK, 16), lambda i: (0, i, 0)),
            pl.BlockSpec((1, din), lambda i: (0, 0)),
            pl.BlockSpec((16, din), lambda i: (0, 0)),
        ],
        out_specs=pl.BlockSpec((RBLK, din), lambda i: (i, 0)),
        out_shape=jax.ShapeDtypeStruct((N, din), jnp.float32),
    )(P, S, bias, rep)


_GDN = lax.GatherDimensionNumbers(
    offset_dims=(), collapsed_slice_dims=(0,), start_index_map=(0,))


def _lane_splat(v, k):
    idx = jnp.full((16, 1), k, jnp.int32)
    return lax.gather(v, idx, _GDN, (1,),
                      mode=lax.GatherScatterMode.PROMISE_IN_BOUNDS)


def _make_sc_edge(D, heads):
    chunks = D // 16
    mesh = plsc.VectorSubcoreMesh(core_axis_name="c", subcore_axis_name="s",
                                  num_cores=2, num_subcores=16)

    @functools.partial(
        pl.kernel,
        out_type=[
            jax.ShapeDtypeStruct((2, NPAD, D), jnp.float32),
            jax.ShapeDtypeStruct((2, NPAD, 16), jnp.float32),
        ],
        mesh=mesh,
        compiler_params=pltpu.CompilerParams(use_tc_tiling_on_sc=False),
        scratch_types=(
            [pltpu.VMEM((2, B), jnp.int32)] * 4
            + [pltpu.VMEM((B, 16), jnp.float32)] * 4
            + [pltpu.VMEM((B, D), jnp.float32)] * 2
            + [pltpu.VMEM((B, 16), jnp.float32)] * 2
            + [
                pltpu.VMEM((8, D), jnp.float32),
                pltpu.VMEM((8, 16), jnp.float32),
                pltpu.VMEM_SHARED((NPAD, D), jnp.float32),
                pltpu.VMEM_SHARED((NPAD, 16), jnp.float32),
            ]
            + [pltpu.SemaphoreType.DMA] * 8
        ),
    )
    def k(h_hbm, asd_hbm, add_hbm, sd_hbm, p_out, s_out,
          sd0, sd1, sd2, sd3,
          asd0, asd1, add0, add1, h0, h1, ex0, ex1, zb, zbs, acc, sacc,
          gsem0, gsem1, ssem0, ssem1, isem0, isem1, isem2, isem3):
        sd_v = (sd0, sd1, sd2, sd3)
        src_v = tuple(r.at[0] for r in sd_v)
        dst_v = tuple(r.at[1] for r in sd_v)
        asd_v, add_v = (asd0, asd1), (add0, add1)
        h_v, ex_v = (h0, h1), (ex0, ex1)
        gsem, ssem = (gsem0, gsem1), (ssem0, ssem1)
        isem = (isem0, isem1, isem2, isem3)

        c = lax.axis_index("c")
        s = lax.axis_index("s")
        w = c * 16 + s

        zero = jnp.zeros((16,), jnp.float32)
        for i in range(8):
            for j in range(chunks):
                zb[i, pl.ds(j * 16, 16)] = zero
            zbs[i, :] = zero
        r0 = s * RPT

        def zrow(i, carry):
            pltpu.sync_copy(zb, acc.at[pl.ds(r0 + i * 8, 8)])
            pltpu.sync_copy(zbs, sacc.at[pl.ds(r0 + i * 8, 8)])
            return carry

        lax.fori_loop(0, RPT // 8, zrow, 0)
        plsc.subcore_barrier()

        base_w = w * ET

        def issue_idx(r, blk_idx):
            base = base_w + blk_idx * B
            pltpu.async_copy(sd_hbm.at[:, pl.ds(base, B)], sd_v[r], isem[r])

        def wait_idx(r):
            pltpu.make_async_copy(sd_hbm.at[:, pl.ds(0, B)], sd_v[r],
                                  isem[r]).wait()

        def issue_gathers(g, r):
            pltpu.async_copy(asd_hbm.at[src_v[r]], asd_v[g], gsem[g])
            pltpu.async_copy(add_hbm.at[dst_v[r]], add_v[g], gsem[g])
            pltpu.async_copy(h_hbm.at[src_v[r]], h_v[g], gsem[g])

        def wait_gathers(g):
            pltpu.make_async_copy(asd_hbm.at[src_v[0]], asd_v[g],
                                  gsem[g]).wait()
            pltpu.make_async_copy(add_hbm.at[dst_v[0]], add_v[g],
                                  gsem[g]).wait()
            pltpu.make_async_copy(h_hbm.at[src_v[0]], h_v[g], gsem[g]).wait()

        def issue_scatter(g, r):
            pltpu.async_copy(ex_v[g], sacc.at[dst_v[r]], ssem[g], add=True)
            pltpu.async_copy(h_v[g], acc.at[dst_v[r]], ssem[g], add=True)

        def wait_scatter(g):
            pltpu.make_async_copy(ex_v[g], sacc.at[dst_v[0]], ssem[g]).wait()
            pltpu.make_async_copy(h_v[g], acc.at[dst_v[0]], ssem[g]).wait()

        issue_idx(0, jnp.int32(0))
        wait_idx(0)
        issue_idx(1, jnp.int32(1))
        issue_gathers(0, 0)
        NB4 = NBLK // 4

        def blk4(jj, carry):
            for p in (0, 1, 2, 3):
                j = 4 * jj + p
                g = p % 2
                o = 1 - g
                r_cur = p
                r_nxt = (p + 1) % 4
                r_pf = (p + 2) % 4

                if p == 0:
                    @pl.when(jj >= 1)
                    def _():
                        wait_scatter(o)
                else:
                    wait_scatter(o)
                if p >= 2:
                    @pl.when(jj < NB4 - 1)
                    def _():
                        issue_idx(r_pf, j + 2)
                else:
                    issue_idx(r_pf, j + 2)
                if p == 3:
                    @pl.when(jj < NB4 - 1)
                    def _():
                        wait_idx(r_nxt)
                        issue_gathers(o, r_nxt)
                else:
                    wait_idx(r_nxt)
                    issue_gathers(o, r_nxt)
                wait_gathers(g)

                @plsc.parallel_loop(0, B, unroll=2)
                def edge(b):
                    e = asd_v[g][b, :] + add_v[g][b, :]
                    ex = jnp.exp(jnp.maximum(e, 0.2 * e))
                    ex_v[g][b, :] = ex
                    for kk in range(chunks):
                        if heads == 8:
                            bc = _lane_splat(ex, kk)
                        else:
                            bc = ex
                        h_v[g][b, pl.ds(kk * 16, 16)] = (
                            h_v[g][b, pl.ds(kk * 16, 16)] * bc)

                issue_scatter(g, r_cur)
            return carry

        lax.fori_loop(0, NB4, blk4, 0)
        wait_scatter(1)
        plsc.subcore_barrier()
        pltpu.sync_copy(acc.at[pl.ds(r0, RPT)], p_out.at[c, pl.ds(r0, RPT)])
        pltpu.sync_copy(sacc.at[pl.ds(r0, RPT)], s_out.at[c, pl.ds(r0, RPT)])

    return k


_sc_edge_128 = _make_sc_edge(128, 8)
_sc_edge_64 = _make_sc_edge(64, 1)


_Z8 = np.zeros((128, 16), np.float32)
for _k in range(8):
    for _c in range(16):
        _Z8[_k * 16 + _c, _k] = 1.0
        _Z8[_k * 16 + _c, _k + 8] = 1.0
_Z1 = np.ones((64, 16), np.float32)

_REP8 = np.zeros((16, 128), np.float32)
for _k in range(8):
    _REP8[_k, _k * 16:(_k + 1) * 16] = 0.5
    _REP8[_k + 8, _k * 16:(_k + 1) * 16] = 0.5
_REP1 = np.full((16, 64), 1.0 / 16.0, np.float32)


def _fold(W, a_src, a_dst, z):
    ms = z * a_src.reshape(-1)[:, None]
    md = z * a_dst.reshape(-1)[:, None]
    return jnp.concatenate([W, W @ ms, W @ md], axis=1)


def kernel(x, edge_index, W1, a_src1, a_dst1, b1, W2, a_src2, a_dst2, b2,
           W3, a_src3, a_dst3, b3):
    loop = jnp.arange(N, dtype=jnp.int32)
    padi = jnp.full((EPAD - E - N,), N, jnp.int32)
    srcp = jnp.concatenate([edge_index[0], loop, padi])
    dstp = jnp.concatenate([edge_index[1], loop, padi])
    sd = jnp.stack([srcp, dstp])
    xpad = jnp.pad(x, ((0, NPAD - N), (0, 0)))

    H, ASD, ADD = _tc_front(xpad, _fold(W1, a_src1, a_dst1, _Z8), 128)
    P, S = _sc_edge_128(H, ASD, ADD, sd)
    H, ASD, ADD = _tc_mid(P, S, b1.reshape(1, -1),
                          _fold(W2, a_src2, a_dst2, _Z8), _REP8, 128, 128)
    P, S = _sc_edge_128(H, ASD, ADD, sd)
    H, ASD, ADD = _tc_mid(P, S, b2.reshape(1, -1),
                          _fold(W3, a_src3, a_dst3, _Z1), _REP8, 128, 64)
    P, S = _sc_edge_64(H, ASD, ADD, sd)
    return _tc_fin(P, S, b3.reshape(1, -1), _REP1, 64)
